# baseline (speedup 1.0000x reference)
import functools

import jax
import jax.numpy as jnp
from jax import lax
from jax.experimental import pallas as pl
from jax.experimental.pallas import tpu as pltpu
from jax.experimental.pallas import tpu_sc as plsc

CHUNK = 128


@functools.lru_cache(maxsize=None)
def _make_kernel(B, V, D):
    info = plsc.get_sparse_core_info()
    NC, NS = info.num_cores, info.num_subcores
    NW = NC * NS
    assert B % (NW * CHUNK) == 0
    b_per_w = B // NW
    n_chunks = b_per_w // CHUNK

    mesh = plsc.VectorSubcoreMesh(core_axis_name="c", subcore_axis_name="s")

    @functools.partial(
        pl.kernel,
        mesh=mesh,
        compiler_params=pltpu.CompilerParams(use_tc_tiling_on_sc=False),
        out_type=(
            jax.ShapeDtypeStruct((B, D), jnp.float32),
            jax.ShapeDtypeStruct((B, D), jnp.float32),
        ),
        scratch_types=[
            pltpu.VMEM((n_chunks, CHUNK), jnp.int32),
            pltpu.VMEM((b_per_w, D), jnp.float32),
            pltpu.VMEM((b_per_w, D), jnp.float32),
            pltpu.SemaphoreType.DMA,
            pltpu.SemaphoreType.DMA,
        ],
    )
    def k(ids_hbm, tshape_hbm, tapp_hbm, out_s_hbm, out_a_hbm,
          idx_v, rows_s, rows_a, sem_s, sem_a):
        wid = lax.axis_index("s") * NC + lax.axis_index("c")
        base = wid * b_per_w
        pltpu.sync_copy(ids_hbm.at[pl.ds(wid * n_chunks, n_chunks)], idx_v)
        copies_s = []
        copies_a = []
        for j in range(n_chunks):
            dst = pl.ds(j * CHUNK, CHUNK)
            copies_s.append(
                pltpu.async_copy(tshape_hbm.at[idx_v.at[j]], rows_s.at[dst], sem_s))
            copies_a.append(
                pltpu.async_copy(tapp_hbm.at[idx_v.at[j]], rows_a.at[dst], sem_a))
        for c in copies_s:
            c.wait()
        pltpu.sync_copy(rows_s, out_s_hbm.at[pl.ds(base, b_per_w)])
        for c in copies_a:
            c.wait()
        pltpu.sync_copy(rows_a, out_a_hbm.at[pl.ds(base, b_per_w)])

    return k


def kernel(instance_ids, table_shape, table_appearance):
    ids = jnp.squeeze(instance_ids)
    B = ids.shape[0]
    V, D = table_shape.shape
    ids2 = ids.reshape(B // CHUNK, CHUNK)
    k = _make_kernel(B, V, D)
    return k(ids2, table_shape, table_appearance)


# speedup vs baseline: 4.6844x; 4.6844x over previous
import functools

import jax
import jax.numpy as jnp
from jax import lax
from jax.experimental import pallas as pl
from jax.experimental.pallas import tpu as pltpu
from jax.experimental.pallas import tpu_sc as plsc

L = 16


@functools.lru_cache(maxsize=None)
def _make_kernel(B, V, D):
    info = plsc.get_sparse_core_info()
    NC, NS = info.num_cores, info.num_subcores
    NW = NC * NS
    NCOL = (V + 127) // 128
    CPW = (NCOL + NW - 1) // NW
    NBKT = CPW + 1
    CPAD = ((NBKT + 1 + L - 1) // L + 1) * L
    EPAD = B + L
    NVREG = B // L
    assert B % L == 0 and D == 64

    mesh = plsc.VectorSubcoreMesh(core_axis_name="c", subcore_axis_name="s")

    @functools.partial(
        pl.kernel,
        mesh=mesh,
        compiler_params=pltpu.CompilerParams(needs_layout_passes=False),
        out_type=(
            jax.ShapeDtypeStruct((B * D,), jnp.float32),
            jax.ShapeDtypeStruct((B * D,), jnp.float32),
        ),
        scratch_types=[
            pltpu.VMEM((EPAD,), jnp.int32),
            pltpu.VMEM((EPAD,), jnp.int32),
            pltpu.VMEM((CPAD,), jnp.int32),
            pltpu.VMEM((CPAD,), jnp.int32),
            pltpu.VMEM((CPAD,), jnp.int32),
            pltpu.VMEM((5, D, 128), jnp.float32),
            pltpu.VMEM((5, D, 128), jnp.float32),
            pltpu.VMEM((8 * D,), jnp.float32),
            pltpu.VMEM((8 * D,), jnp.float32),
            pltpu.SemaphoreType.DMA,
            pltpu.SemaphoreType.DMA,
            pltpu.SemaphoreType.DMA,
            pltpu.SemaphoreType.DMA,
            pltpu.SemaphoreType.DMA,
            pltpu.SemaphoreType.DMA,
            pltpu.SemaphoreType.DMA,
        ],
    )
    def k(ids_hbm, ts_hbm, ta_hbm, tail_s_hbm, tail_a_hbm, o_s_hbm, o_a_hbm,
          ids_v, ent, counts, offs, run,
          cb_s, cb_a, stage_s, stage_a,
          fsem0, fsem1, fsem2, fsem3, fsem4, wsem_s, wsem_a):
        lane = lax.iota(jnp.int32, L)
        wid = lax.axis_index("s") * NC + lax.axis_index("c")
        lo = wid * CPW
        hi = jnp.minimum(lo + CPW, NCOL)
        ncw = hi - lo

        last_col = NCOL - 1
        NR = 5
        fsems = (fsem0, fsem1, fsem2, fsem3, fsem4)

        def issue_ci(sub, ci):
            gc = lo + ci

            @pl.when(gc == last_col)
            def _():
                pltpu.async_copy(tail_s_hbm, cb_s.at[sub], fsems[sub])
                pltpu.async_copy(tail_a_hbm, cb_a.at[sub], fsems[sub])

            @pl.when(gc != last_col)
            def _():
                off = pl.multiple_of(gc * 128, 128)
                for h in range(2):
                    hs = pl.ds(h * 32, 32)
                    pltpu.async_copy(ts_hbm.at[hs, pl.ds(off, 128)],
                                     cb_s.at[sub].at[hs], fsems[sub])
                    pltpu.async_copy(ta_hbm.at[hs, pl.ds(off, 128)],
                                     cb_a.at[sub].at[hs], fsems[sub])

        for sub in range(NR):
            @pl.when(sub < ncw)
            def _(sub=sub):
                issue_ci(sub, sub)

        pltpu.sync_copy(ids_hbm.at[pl.ds(0, B)], ids_v.at[pl.ds(0, B)])

        def compact_body(i, ne):
            v = ids_v[pl.ds(i * L, L)]
            col = v >> 7
            m = (col >= lo) & (col < hi)
            e = ((col - lo) << 21) | ((i * L + lane) << 7) | (v & 127)
            plsc.store_compressed(ent.at[pl.ds(ne, L)], e, mask=m)
            return ne + plsc.all_reduce_population_count(m)[0]

        ne = lax.fori_loop(0, NVREG, compact_body, 0)
        nv = (ne + L - 1) // L

        zeros = jnp.zeros((L,), jnp.int32)
        for i in range(CPAD // L):
            counts[pl.ds(i * L, L)] = zeros
        m0 = lane == 0

        def sstore(ref, at, val):
            plsc.store_scatter(ref, [jnp.full((L,), at, jnp.int32)],
                               jnp.full((L,), val, jnp.int32), mask=m0)

        def count_body(i, carry):
            v = ent[pl.ds(i * L, L)]
            b = jnp.where(i * L + lane < ne, v >> 21, CPW)
            for l in range(L):
                bl = b[l]
                cv = counts[pl.ds(bl, L)]
                sstore(counts, bl, cv[0] + 1)
            return carry

        lax.fori_loop(0, nv, count_body, 0)

        sstore(offs, 0, 0)

        def prefix_body(c, acc):
            cv = counts[pl.ds(c, L)]
            acc = acc + cv[0]
            sstore(offs, c + 1, acc)
            return acc

        lax.fori_loop(0, NBKT, prefix_body, 0)
        for i in range(CPAD // L):
            run[pl.ds(i * L, L)] = offs[pl.ds(i * L, L)]

        def place_body(i, carry):
            v = ent[pl.ds(i * L, L)]
            b = jnp.where(i * L + lane < ne, v >> 21, CPW)
            packed = v & 0x1FFFFF
            for l in range(L):
                bl = b[l]
                slot = run[pl.ds(bl, L)][0]
                sstore(run, bl, slot + 1)
                sstore(ids_v, slot, packed[l])
            return carry

        lax.fori_loop(0, nv, place_body, 0)

        def nz_body(i, nnz):
            cvec = counts[pl.ds(i * L, L)]
            m = (cvec > 0) & (i * L + lane < ncw)
            plsc.store_compressed(counts.at[pl.ds(nnz, L)], i * L + lane, mask=m)
            return nnz + plsc.all_reduce_population_count(m)[0]

        nnz = lax.fori_loop(0, (CPW + L - 1) // L, nz_body, 0)

        def wait_fetch(sub):
            pltpu.make_async_copy(
                ts_hbm.at[:, pl.ds(0, 128)], cb_s.at[sub], fsems[sub]).wait()
            pltpu.make_async_copy(
                ta_hbm.at[:, pl.ds(0, 128)], cb_a.at[sub], fsems[sub]).wait()

        def process_ci(sub, ci):
            ov = offs[pl.ds(ci, L)]
            o0, o1 = ov[0], ov[1]

            def emit(e, carry):
                j = o0 + e
                w = ids_v[pl.ds(j, L)][0]
                wrl = w & 127
                wpos = w >> 7
                ss = (j & 7) * D

                @pl.when(j >= 8)
                def _():
                    pltpu.make_async_copy(
                        stage_s.at[pl.ds(0, D)], o_s_hbm.at[pl.ds(0, D)],
                        wsem_s).wait()
                    pltpu.make_async_copy(
                        stage_a.at[pl.ds(0, D)], o_a_hbm.at[pl.ds(0, D)],
                        wsem_a).wait()

                rlv = jnp.full((L,), wrl, jnp.int32)
                for kk in range(D // L):
                    dsub = lane + kk * L
                    stage_s[pl.ds(ss + kk * L, L)] = plsc.load_gather(
                        cb_s.at[sub], [dsub, rlv])
                    stage_a[pl.ds(ss + kk * L, L)] = plsc.load_gather(
                        cb_a.at[sub], [dsub, rlv])
                pltpu.async_copy(
                    stage_s.at[pl.ds(ss, D)],
                    o_s_hbm.at[pl.ds(wpos * D, D)], wsem_s)
                pltpu.async_copy(
                    stage_a.at[pl.ds(ss, D)],
                    o_a_hbm.at[pl.ds(wpos * D, D)], wsem_a)
                return carry

            lax.fori_loop(0, o1 - o0, emit, 0)

        def issue_nz(sub, k):
            issue_ci(sub, counts[pl.ds(k, L)][0])

        nzv = counts[pl.ds(0, L)]
        k_start = plsc.all_reduce_population_count(
            (nzv < NR) & (lane < nnz))[0]

        for sub in range(NR):
            @pl.when(sub < ncw)
            def _(sub=sub):
                wait_fetch(sub)
                process_ci(sub, sub)

                @pl.when(k_start + sub < nnz)
                def _(sub=sub):
                    issue_nz(sub, k_start + sub)

        def quint_body(q, carry):
            for sub in range(NR):
                k = k_start + NR * q + sub

                @pl.when(k < nnz)
                def _(sub=sub, k=k):
                    wait_fetch(sub)
                    process_ci(sub, counts[pl.ds(k, L)][0])

                    @pl.when(k + NR < nnz)
                    def _(sub=sub, k=k):
                        issue_nz(sub, k + NR)

            return carry

        lax.fori_loop(0, (nnz - k_start + NR - 1) // NR, quint_body, 0)

        def drain_body(kk, carry):
            pltpu.make_async_copy(
                stage_s.at[pl.ds(0, D)], o_s_hbm.at[pl.ds(0, D)], wsem_s).wait()
            pltpu.make_async_copy(
                stage_a.at[pl.ds(0, D)], o_a_hbm.at[pl.ds(0, D)], wsem_a).wait()
            return carry

        lax.fori_loop(0, jnp.minimum(ne, 8), drain_body, 0)

    return k


def kernel(instance_ids, table_shape, table_appearance):
    ids = jnp.squeeze(instance_ids)
    B = ids.shape[0]
    V, D = table_shape.shape
    tail_rows = ((V + 127) // 128 - 1) * 128
    npad = 128 - (V - tail_rows)

    def tail(t):
        return jnp.pad(t[tail_rows:, :].T, ((0, 0), (0, npad)))

    k = _make_kernel(B, V, D)
    o_s, o_a = k(ids, table_shape.T, table_appearance.T,
                 tail(table_shape), tail(table_appearance))
    return (o_s.reshape(B, D), o_a.reshape(B, D))
